# baseline (speedup 1.0000x reference)
import jax
import jax.numpy as jnp
from jax.experimental import pallas as pl

_BLOCK_S = 2048


def _body(x_ref, pos_ref, out_ref):
    out_ref[...] = x_ref[...] + pos_ref[...][None]


def kernel(x, pos_table):
    B, S, D = x.shape
    grid = (S // _BLOCK_S, B)
    return pl.pallas_call(
        _body,
        grid=grid,
        in_specs=[
            pl.BlockSpec((1, _BLOCK_S, D), lambda i, b: (b, i, 0)),
            pl.BlockSpec((_BLOCK_S, D), lambda i, b: (i, 0)),
        ],
        out_specs=pl.BlockSpec((1, _BLOCK_S, D), lambda i, b: (b, i, 0)),
        out_shape=jax.ShapeDtypeStruct((B, S, D), x.dtype),
    )(x, pos_table)
